# baseline (speedup 1.0000x reference)
import jax
import jax.numpy as jnp
from jax import lax
from jax.experimental import pallas as pl
from jax.experimental.pallas import tpu as pltpu

_N_Q = 8
_N_CODES = 1024
_D = 32
_AUG = 40
_NH = 4


def _gather_rows(out):
    q = out[0:_D] + out[_AUG:_AUG + _D] + out[2 * _AUG:2 * _AUG + _D]
    idxrow = (out[_D:_D + 1] + out[_AUG + _D:_AUG + _D + 1]
              + out[2 * _AUG + _D:2 * _AUG + _D + 1])
    cnt = (out[_D + 1:_D + 2] + out[_AUG + _D + 1:_AUG + _D + 2]
           + out[2 * _AUG + _D + 1:2 * _AUG + _D + 2])
    return q, idxrow, cnt


def _rqv_body(x_ref, w_ref, wall_ref, y_ref, idx_ref, sq_ref):
    s = x_ref.shape[2]
    s2 = s // _NH
    dn = (((1,), (0,)), ((), ()))
    sqtot = jnp.float32(0.0)
    flagv = jnp.zeros((1, s2), jnp.float32)
    for h in range(_NH):
        cols = slice(h * s2, (h + 1) * s2)
        xh = x_ref[0, :, cols]
        res = xh
        acc = jnp.zeros_like(xh)
        for i in range(_N_Q):
            w = w_ref[i]
            w2 = w + w
            c2 = jnp.sum(w * w, axis=1, keepdims=True)
            r2 = jnp.sum(res * res, axis=0, keepdims=True)
            scores2 = lax.dot_general(
                w2, res, dn,
                preferred_element_type=jnp.float32,
                precision=lax.Precision.DEFAULT)
            obj = (r2 + c2) - scores2
            mn = jnp.min(obj, axis=0, keepdims=True)
            oh = jnp.where(obj == mn, 1.0, 0.0).astype(jnp.bfloat16)
            out = lax.dot_general(wall_ref[i], oh, dn,
                                  preferred_element_type=jnp.float32)
            q, idxrow, cnt = _gather_rows(out)
            flagv = jnp.maximum(flagv, cnt)
            idx_ref[0, pl.ds(i, 1), cols] = idxrow.astype(jnp.int32)
            acc = acc + q
            res = res - q
        y_ref[0, :, cols] = acc
        diff = acc - xh
        sqtot = sqtot + jnp.sum(diff * diff)
    sq_ref[0] = jnp.full((1, 128), sqtot, dtype=jnp.float32)

    @pl.when(jnp.max(flagv) > 1.5)
    def _redo_with_ties():
        x = x_ref[0]
        res = x
        acc = jnp.zeros_like(x)
        for i in range(_N_Q):
            w = w_ref[i]
            w2 = w + w
            c2 = jnp.sum(w * w, axis=1, keepdims=True)
            r2 = jnp.sum(res * res, axis=0, keepdims=True)
            scores2 = lax.dot_general(
                w2, res, dn,
                preferred_element_type=jnp.float32,
                precision=lax.Precision.DEFAULT)
            obj = (r2 + c2) - scores2
            mn = jnp.min(obj, axis=0, keepdims=True)
            hitb = obj == mn
            iota = lax.broadcasted_iota(jnp.int32, (_N_CODES, s), 0)
            idx2 = jnp.min(jnp.where(hitb, iota, _N_CODES),
                           axis=0, keepdims=True)
            oh = jnp.where(iota == idx2, 1.0, 0.0).astype(jnp.bfloat16)
            out = lax.dot_general(wall_ref[i], oh, dn,
                                  preferred_element_type=jnp.float32)
            q, _, _ = _gather_rows(out)
            idx_ref[0, pl.ds(i, 1), :] = idx2
            acc = acc + q
            res = res - q
        y_ref[0] = acc
        diff = acc - x
        sq_ref[0] = jnp.full((1, 128), jnp.sum(diff * diff), dtype=jnp.float32)


def kernel(data_object, weights, N_i, m_i):
    b, d, s = data_object.shape
    wt = jnp.transpose(weights, (0, 2, 1))
    iota = jnp.broadcast_to(
        jnp.arange(_N_CODES, dtype=jnp.float32)[None, None, :],
        (_N_Q, 1, _N_CODES))
    ones = jnp.ones((_N_Q, 1, _N_CODES), jnp.float32)
    zpad = jnp.zeros((_N_Q, _AUG - _D - 2, _N_CODES), jnp.float32)
    waug = jnp.concatenate([wt, iota, ones, zpad], axis=1)

    def mask16(v):
        bits = lax.bitcast_convert_type(v, jnp.int32)
        return lax.bitcast_convert_type(
            jnp.bitwise_and(bits, jnp.int32(-65536)), jnp.float32)

    h1 = mask16(waug)
    r1 = waug - h1
    h2 = mask16(r1)
    r2 = r1 - h2
    wall = jnp.concatenate([h1.astype(jnp.bfloat16),
                            h2.astype(jnp.bfloat16),
                            r2.astype(jnp.bfloat16)], axis=1)

    grid = (b,)
    y, idx, sq = pl.pallas_call(
        _rqv_body,
        grid=grid,
        in_specs=[
            pl.BlockSpec((1, d, s), lambda i: (i, 0, 0)),
            pl.BlockSpec((_N_Q, _N_CODES, _D), lambda i: (0, 0, 0)),
            pl.BlockSpec((_N_Q, 3 * _AUG, _N_CODES), lambda i: (0, 0, 0)),
        ],
        out_specs=[
            pl.BlockSpec((1, d, s), lambda i: (i, 0, 0)),
            pl.BlockSpec((1, _N_Q, s), lambda i: (i, 0, 0)),
            pl.BlockSpec((1, 1, 128), lambda i: (i, 0, 0)),
        ],
        out_shape=[
            jax.ShapeDtypeStruct((b, d, s), jnp.float32),
            jax.ShapeDtypeStruct((b, _N_Q, s), jnp.int32),
            jax.ShapeDtypeStruct((b, 1, 128), jnp.float32),
        ],
        compiler_params=pltpu.CompilerParams(
            dimension_semantics=("arbitrary",),
        ),
    )(data_object, weights, wall)
    commitment_loss = jnp.sum(sq[:, 0, 0]) / (b * d * s)
    return y, commitment_loss, jnp.transpose(idx, (1, 0, 2))


# speedup vs baseline: 2.2388x; 2.2388x over previous
import jax
import jax.numpy as jnp
from jax import lax
from jax.experimental import pallas as pl
from jax.experimental.pallas import tpu as pltpu

_N_Q = 8
_N_CODES = 1024
_D = 32
_AUG = 40
_NH = 2


def _gather_rows(out):
    q = out[0:_D] + out[_AUG:_AUG + _D] + out[2 * _AUG:2 * _AUG + _D]
    idxrow = (out[_D:_D + 1] + out[_AUG + _D:_AUG + _D + 1]
              + out[2 * _AUG + _D:2 * _AUG + _D + 1])
    cnt = (out[_D + 1:_D + 2] + out[_AUG + _D + 1:_AUG + _D + 2]
           + out[2 * _AUG + _D + 1:2 * _AUG + _D + 2])
    return q, idxrow, cnt


def _rqv_body(x_ref, w_ref, wall_ref, y_ref, idx_ref, sq_ref):
    s = x_ref.shape[2]
    s2 = s // _NH
    dn = (((1,), (0,)), ((), ()))
    sqtot = jnp.float32(0.0)
    flagv = jnp.zeros((1, s2), jnp.float32)
    for h in range(_NH):
        cols = slice(h * s2, (h + 1) * s2)
        xh = x_ref[0, :, cols]
        res = xh
        acc = jnp.zeros_like(xh)
        for i in range(_N_Q):
            w = w_ref[i]
            w2 = w + w
            c2 = jnp.sum(w * w, axis=1, keepdims=True)
            r2 = jnp.sum(res * res, axis=0, keepdims=True)
            scores2 = lax.dot_general(
                w2, res, dn,
                preferred_element_type=jnp.float32,
                precision=lax.Precision.DEFAULT)
            obj = (r2 + c2) - scores2
            mn = jnp.min(obj, axis=0, keepdims=True)
            oh = jnp.where(obj == mn, 1.0, 0.0).astype(jnp.bfloat16)
            out = lax.dot_general(wall_ref[i], oh, dn,
                                  preferred_element_type=jnp.float32)
            q, idxrow, cnt = _gather_rows(out)
            flagv = jnp.maximum(flagv, cnt)
            idx_ref[0, pl.ds(i, 1), cols] = idxrow.astype(jnp.int32)
            acc = acc + q
            res = res - q
        y_ref[0, :, cols] = acc
        diff = acc - xh
        sqtot = sqtot + jnp.sum(diff * diff)
    sq_ref[0] = jnp.full((1, 128), sqtot, dtype=jnp.float32)

    @pl.when(jnp.max(flagv) > 1.5)
    def _redo_with_ties():
        x = x_ref[0]
        res = x
        acc = jnp.zeros_like(x)
        for i in range(_N_Q):
            w = w_ref[i]
            w2 = w + w
            c2 = jnp.sum(w * w, axis=1, keepdims=True)
            r2 = jnp.sum(res * res, axis=0, keepdims=True)
            scores2 = lax.dot_general(
                w2, res, dn,
                preferred_element_type=jnp.float32,
                precision=lax.Precision.DEFAULT)
            obj = (r2 + c2) - scores2
            mn = jnp.min(obj, axis=0, keepdims=True)
            hitb = obj == mn
            iota = lax.broadcasted_iota(jnp.int32, (_N_CODES, s), 0)
            idx2 = jnp.min(jnp.where(hitb, iota, _N_CODES),
                           axis=0, keepdims=True)
            oh = jnp.where(iota == idx2, 1.0, 0.0).astype(jnp.bfloat16)
            out = lax.dot_general(wall_ref[i], oh, dn,
                                  preferred_element_type=jnp.float32)
            q, _, _ = _gather_rows(out)
            idx_ref[0, pl.ds(i, 1), :] = idx2
            acc = acc + q
            res = res - q
        y_ref[0] = acc
        diff = acc - x
        sq_ref[0] = jnp.full((1, 128), jnp.sum(diff * diff), dtype=jnp.float32)


def kernel(data_object, weights, N_i, m_i):
    b, d, s = data_object.shape
    wt = jnp.transpose(weights, (0, 2, 1))
    iota = jnp.broadcast_to(
        jnp.arange(_N_CODES, dtype=jnp.float32)[None, None, :],
        (_N_Q, 1, _N_CODES))
    ones = jnp.ones((_N_Q, 1, _N_CODES), jnp.float32)
    zpad = jnp.zeros((_N_Q, _AUG - _D - 2, _N_CODES), jnp.float32)
    waug = jnp.concatenate([wt, iota, ones, zpad], axis=1)

    def mask16(v):
        bits = lax.bitcast_convert_type(v, jnp.int32)
        return lax.bitcast_convert_type(
            jnp.bitwise_and(bits, jnp.int32(-65536)), jnp.float32)

    h1 = mask16(waug)
    r1 = waug - h1
    h2 = mask16(r1)
    r2 = r1 - h2
    wall = jnp.concatenate([h1.astype(jnp.bfloat16),
                            h2.astype(jnp.bfloat16),
                            r2.astype(jnp.bfloat16)], axis=1)

    grid = (b,)
    y, idx, sq = pl.pallas_call(
        _rqv_body,
        grid=grid,
        in_specs=[
            pl.BlockSpec((1, d, s), lambda i: (i, 0, 0)),
            pl.BlockSpec((_N_Q, _N_CODES, _D), lambda i: (0, 0, 0)),
            pl.BlockSpec((_N_Q, 3 * _AUG, _N_CODES), lambda i: (0, 0, 0)),
        ],
        out_specs=[
            pl.BlockSpec((1, d, s), lambda i: (i, 0, 0)),
            pl.BlockSpec((1, _N_Q, s), lambda i: (i, 0, 0)),
            pl.BlockSpec((1, 1, 128), lambda i: (i, 0, 0)),
        ],
        out_shape=[
            jax.ShapeDtypeStruct((b, d, s), jnp.float32),
            jax.ShapeDtypeStruct((b, _N_Q, s), jnp.int32),
            jax.ShapeDtypeStruct((b, 1, 128), jnp.float32),
        ],
        compiler_params=pltpu.CompilerParams(
            dimension_semantics=("arbitrary",),
        ),
    )(data_object, weights, wall)
    commitment_loss = jnp.sum(sq[:, 0, 0]) / (b * d * s)
    return y, commitment_loss, jnp.transpose(idx, (1, 0, 2))
